# Initial kernel scaffold; baseline (speedup 1.0000x reference)
import jax
import jax.numpy as jnp
from jax.experimental import pallas as pl


def kernel(x, lat, lon, mesh_pos, Wt, bt, Wp, bp, Wq, bq, Wk, bk, Wv, bv, gamma, beta):
    raise NotImplementedError("write your pallas kernel here")


# speedup vs baseline: 1.8739x; 1.8739x over previous
import functools

import jax
import jax.numpy as jnp
from jax import lax
from jax.experimental import pallas as pl
from jax.experimental.pallas import tpu as pltpu
from jax.experimental.pallas import tpu_sc as plsc

_K = 16
_CHUNK = 2048
_BIGF = 1e30
_BIGI = 2**30


def _knn_body(mesh_ref, pos_ref, m2_ref, p2_ref, dval_ref, didx_ref,
              run_val, run_idx, loc_val, loc_idx, *, ntok, mpad):
    step = pl.program_id(0)
    nsteps = pl.num_programs(0)

    @pl.when(step == 0)
    def _init():
        run_val[...] = jnp.full((mpad, _K), _BIGF, jnp.float32)
        run_idx[...] = jnp.zeros((mpad, _K), jnp.int32)

    mesh = mesh_ref[...]
    posb = pos_ref[...]
    m2 = m2_ref[...]
    p2 = p2_ref[...]
    dot = jnp.dot(mesh, posb)
    d2 = (m2 + p2) - 2.0 * dot
    dist = jnp.sqrt(jnp.maximum(d2, 1e-12))
    gidx = step * _CHUNK + lax.broadcasted_iota(jnp.int32, (mpad, _CHUNK), 1)
    dist = jnp.where(gidx < ntok, dist, _BIGF)

    for t in range(_K):
        minval = jnp.min(dist, axis=1, keepdims=True)
        cand = jnp.where(dist == minval, gidx, _BIGI)
        amin = jnp.min(cand, axis=1, keepdims=True)
        loc_val[:, t:t + 1] = minval
        loc_idx[:, t:t + 1] = amin
        dist = jnp.where(gidx == amin, _BIGF, dist)

    cv = jnp.concatenate([run_val[...], loc_val[...]], axis=1)
    ci = jnp.concatenate([run_idx[...], loc_idx[...]], axis=1)
    for t in range(_K):
        minval = jnp.min(cv, axis=1, keepdims=True)
        cand = jnp.where(cv == minval, ci, _BIGI)
        amin = jnp.min(cand, axis=1, keepdims=True)
        run_val[:, t:t + 1] = minval
        run_idx[:, t:t + 1] = amin
        cv = jnp.where(ci == amin, _BIGF, cv)

    @pl.when(step == nsteps - 1)
    def _emit():
        dval_ref[...] = run_val[...]
        didx_ref[...] = run_idx[...]


def _knn_topk(mesh_pad, pos_pad, m2_pad, p2_pad, ntok, mpad, npad):
    grid = npad // _CHUNK
    return pl.pallas_call(
        functools.partial(_knn_body, ntok=ntok, mpad=mpad),
        grid=(grid,),
        in_specs=[
            pl.BlockSpec((mpad, 3), lambda i: (0, 0)),
            pl.BlockSpec((3, _CHUNK), lambda i: (0, i)),
            pl.BlockSpec((mpad, 1), lambda i: (0, 0)),
            pl.BlockSpec((1, _CHUNK), lambda i: (0, i)),
        ],
        out_specs=[
            pl.BlockSpec((mpad, _K), lambda i: (0, 0)),
            pl.BlockSpec((mpad, _K), lambda i: (0, 0)),
        ],
        out_shape=[
            jax.ShapeDtypeStruct((mpad, _K), jnp.float32),
            jax.ShapeDtypeStruct((mpad, _K), jnp.int32),
        ],
        scratch_shapes=[
            pltpu.VMEM((mpad, _K), jnp.float32),
            pltpu.VMEM((mpad, _K), jnp.int32),
            pltpu.VMEM((mpad, _K), jnp.float32),
            pltpu.VMEM((mpad, _K), jnp.int32),
        ],
    )(mesh_pad, pos_pad, m2_pad, p2_pad)


def _sc_gather(tok_table, pos_table, tok_idx, pos_idx):
    rows = tok_idx.shape[0]
    cdim = tok_table.shape[1]
    pdim = pos_table.shape[1]
    info = plsc.get_sparse_core_info()
    nc, ns = info.num_cores, info.num_subcores
    nw = nc * ns
    assert rows % (8 * nw) == 0 and cdim == pdim
    per_w = rows // nw
    mesh = plsc.VectorSubcoreMesh(core_axis_name="c", subcore_axis_name="s")

    @functools.partial(
        pl.kernel, mesh=mesh,
        out_type=(
            jax.ShapeDtypeStruct((rows, cdim), jnp.float32),
            jax.ShapeDtypeStruct((rows, pdim), jnp.float32),
        ),
        scratch_types=[
            pltpu.VMEM((per_w,), jnp.int32),
            pltpu.VMEM((per_w, cdim), jnp.float32),
            pltpu.SemaphoreType.DMA,
        ],
    )
    def gk(tok_hbm, pos_hbm, ti_hbm, pi_hbm, otok_hbm, opos_hbm,
           idx_v, rows_v, sem):
        wid = lax.axis_index("s") * nc + lax.axis_index("c")
        base = wid * per_w
        pltpu.sync_copy(ti_hbm.at[pl.ds(base, per_w)], idx_v)
        pltpu.async_copy(tok_hbm.at[idx_v], rows_v, sem).wait()
        pltpu.sync_copy(rows_v, otok_hbm.at[pl.ds(base, per_w)])
        pltpu.sync_copy(pi_hbm.at[pl.ds(base, per_w)], idx_v)
        pltpu.async_copy(pos_hbm.at[idx_v], rows_v, sem).wait()
        pltpu.sync_copy(rows_v, opos_hbm.at[pl.ds(base, per_w)])

    return gk(tok_table, pos_table, tok_idx, pos_idx)


def _attn_body(tg_ref, pg_ref, dist_ref, mesh_ref,
               Wt_ref, Wp_ref, Wq_ref, Wk_ref, Wv_ref,
               bt_ref, bp_ref, bq_ref, bk_ref, bv_ref,
               gamma_ref, beta_ref, out_ref, *, dmodel):
    f32 = jnp.float32
    Wt = Wt_ref[...]
    Wp = Wp_ref[...]
    Wq = Wq_ref[...]
    Wk = Wk_ref[...]
    Wv = Wv_ref[...]
    btp = bt_ref[...] + bp_ref[...]
    A_k = jnp.dot(Wt, Wk, preferred_element_type=f32)
    A_v = jnp.dot(Wt, Wv, preferred_element_type=f32)
    B_k = jnp.dot(Wp, Wk, preferred_element_type=f32)
    B_v = jnp.dot(Wp, Wv, preferred_element_type=f32)
    c_k = jnp.dot(btp, Wk, preferred_element_type=f32) + bk_ref[...]
    c_v = jnp.dot(btp, Wv, preferred_element_type=f32) + bv_ref[...]

    mesh = mesh_ref[...]
    query = jnp.dot(jnp.dot(mesh, Wp, preferred_element_type=f32) + bp_ref[...],
                    Wq, preferred_element_type=f32) + bq_ref[...]
    qA = lax.dot_general(query, A_k, (((1,), (1,)), ((), ())),
                         preferred_element_type=f32)
    qB = lax.dot_general(query, B_k, (((1,), (1,)), ((), ())),
                         preferred_element_type=f32)
    qc = jnp.sum(query * c_k, axis=1, keepdims=True)

    cols = []
    for j in range(_K):
        tj = tg_ref[0, :, j, :]
        pj = pg_ref[0, :, j, 0:3]
        lt = jnp.sum(tj * qA, axis=1, keepdims=True)
        lp = jnp.sum(pj * qB, axis=1, keepdims=True)
        cols.append(lt + lp)
    logits = jnp.concatenate(cols, axis=1)
    scale = 1.0 / (dmodel ** 0.5)
    logits = (logits + qc) * scale - dist_ref[...]

    mmax = jnp.max(logits, axis=1, keepdims=True)
    ex = jnp.exp(logits - mmax)
    att = ex / jnp.sum(ex, axis=1, keepdims=True)

    tbar = jnp.zeros_like(qA)
    pbar = jnp.zeros((qA.shape[0], 3), f32)
    for j in range(_K):
        aj = att[:, j:j + 1]
        tbar = tbar + tg_ref[0, :, j, :] * aj
        pbar = pbar + pg_ref[0, :, j, 0:3] * aj

    out = (jnp.dot(tbar, A_v, preferred_element_type=f32)
           + jnp.dot(pbar, B_v, preferred_element_type=f32) + c_v)
    mu = jnp.mean(out, axis=1, keepdims=True)
    ctr = out - mu
    var = jnp.mean(ctr * ctr, axis=1, keepdims=True)
    y = ctr / jnp.sqrt(var + 1e-5) * gamma_ref[...] + beta_ref[...]
    out_ref[0] = y


def _attn_combine(tg4, pg4, knn_dist, mesh_pad,
                  Wt, Wp, Wq, Wk, Wv, bt, bp, bq, bk, bv, gamma, beta):
    B, mpad, _, C = tg4.shape
    D = Wt.shape[1]
    pdim = pg4.shape[3]
    full2 = lambda s: pl.BlockSpec(s, lambda b: (0, 0))
    return pl.pallas_call(
        functools.partial(_attn_body, dmodel=D),
        grid=(B,),
        in_specs=[
            pl.BlockSpec((1, mpad, _K, C), lambda b: (b, 0, 0, 0)),
            pl.BlockSpec((1, mpad, _K, pdim), lambda b: (b, 0, 0, 0)),
            full2((mpad, _K)),
            full2((mpad, 3)),
            full2((C, D)),
            full2((3, D)),
            full2((D, D)),
            full2((D, D)),
            full2((D, D)),
            full2((1, D)),
            full2((1, D)),
            full2((1, D)),
            full2((1, D)),
            full2((1, D)),
            full2((1, D)),
            full2((1, D)),
        ],
        out_specs=pl.BlockSpec((1, mpad, D), lambda b: (b, 0, 0)),
        out_shape=jax.ShapeDtypeStruct((B, mpad, D), jnp.float32),
    )(tg4, pg4, knn_dist, mesh_pad, Wt, Wp, Wq, Wk, Wv,
      bt.reshape(1, D), bp.reshape(1, D), bq.reshape(1, D),
      bk.reshape(1, D), bv.reshape(1, D),
      gamma.reshape(1, D), beta.reshape(1, D))


def kernel(x, lat, lon, mesh_pos, Wt, bt, Wp, bp, Wq, bq, Wk, bk, Wv, bv,
           gamma, beta):
    B, C, H, W = x.shape
    D = Wt.shape[1]
    M = mesh_pos.shape[0]
    N = H * W
    mpad = ((M + 7) // 8) * 8
    npad = ((N + _CHUNK - 1) // _CHUNK) * _CHUNK

    lat_g, lon_g = jnp.meshgrid(lat, lon, indexing="ij")
    pos = jnp.stack([jnp.cos(lat_g) * jnp.cos(lon_g),
                     jnp.cos(lat_g) * jnp.sin(lon_g),
                     jnp.sin(lat_g)], axis=-1).reshape(N, 3)

    mesh_pad = jnp.zeros((mpad, 3), jnp.float32).at[:M].set(mesh_pos)
    pos_pad = jnp.zeros((3, npad), jnp.float32).at[:, :N].set(pos.T)
    m2 = jnp.sum(mesh_pos ** 2, axis=1)
    p2 = jnp.sum(pos ** 2, axis=1)
    m2_pad = jnp.zeros((mpad, 1), jnp.float32).at[:M, 0].set(m2)
    p2_pad = jnp.zeros((1, npad), jnp.float32).at[0, :N].set(p2)

    knn_dist, knn_idx = _knn_topk(mesh_pad, pos_pad, m2_pad, p2_pad,
                                  N, mpad, npad)

    tok_table = jnp.transpose(x, (0, 2, 3, 1)).reshape(B * N, C)
    pos_table = jnp.zeros((N, C), jnp.float32).at[:, :3].set(pos)
    iflat = knn_idx.reshape(-1)
    offs = (jnp.arange(B, dtype=jnp.int32) * N)[:, None]
    tok_idx = (iflat[None, :] + offs).reshape(-1)
    pos_idx = jnp.tile(iflat, (B,))

    tok_rows, pos_rows = _sc_gather(tok_table, pos_table, tok_idx, pos_idx)
    tg4 = tok_rows.reshape(B, mpad, _K, C)
    pg4 = pos_rows.reshape(B, mpad, _K, C)

    out = _attn_combine(tg4, pg4, knn_dist, mesh_pad,
                        Wt, Wp, Wq, Wk, Wv, bt, bp, bq, bk, bv, gamma, beta)
    return (out[:, :M, :], mesh_pos)
